# baseline (speedup 1.0000x reference)
import functools

import jax
import jax.numpy as jnp
from jax import lax
from jax.experimental import pallas as pl
from jax.experimental.pallas import tpu as pltpu
from jax.experimental.pallas import tpu_sc as plsc

_N = 10000
_E = 320000
_NUM_GRAPHS = 256
_NUM_REL = 3
_EP = 327680
_ROWS = _EP // 128
_ACC_ROWS = 30080
_STRIPE = _ACC_ROWS // 16
_EMB_PAD = 32768


def _emb_gather(tbl, idx2d):
    mesh = plsc.VectorSubcoreMesh(core_axis_name="c", subcore_axis_name="s")

    @functools.partial(
        pl.kernel,
        out_type=jax.ShapeDtypeStruct((_EMB_PAD, 32), jnp.float32),
        mesh=mesh,
        compiler_params=pltpu.CompilerParams(use_tc_tiling_on_sc=False),
        scratch_types=[
            pltpu.VMEM((8, 128), jnp.int32),
            pltpu.VMEM((1024, 32), jnp.float32),
            pltpu.SemaphoreType.DMA,
        ],
    )
    def k(tbl_hbm, idx_hbm, out_hbm, idx_v, rows_v, sem):
        cid = lax.axis_index("c")
        sid = lax.axis_index("s")
        wid = sid * 2 + cid
        pltpu.sync_copy(idx_hbm.at[pl.ds(wid * 8, 8)], idx_v)
        gathers = [
            pltpu.async_copy(
                tbl_hbm.at[idx_v.at[j]], rows_v.at[pl.ds(j * 128, 128)], sem
            )
            for j in range(8)
        ]
        for d in gathers:
            d.wait()
        pltpu.sync_copy(rows_v, out_hbm.at[pl.ds(wid * 1024, 1024)])

    return k(tbl, idx2d)


def _edge_agg_pipe(h_stack, gidx, sidx, z_acc, dh, krows, dtype=jnp.float32):
    mesh = plsc.VectorSubcoreMesh(core_axis_name="c", subcore_axis_name="s")
    npairs = 160 // krows // 2

    @functools.partial(
        pl.kernel,
        out_type=jax.ShapeDtypeStruct((2 * _ACC_ROWS, dh), dtype),
        mesh=mesh,
        compiler_params=pltpu.CompilerParams(use_tc_tiling_on_sc=False),
        scratch_types=[
            pltpu.VMEM((krows, 128), jnp.int32),
            pltpu.VMEM((krows, 128), jnp.int32),
            pltpu.VMEM((krows, 128), jnp.int32),
            pltpu.VMEM((krows, 128), jnp.int32),
            pltpu.VMEM((krows * 128, dh), dtype),
            pltpu.VMEM((krows * 128, dh), dtype),
            pltpu.VMEM_SHARED((_ACC_ROWS, dh), dtype),
            pltpu.SemaphoreType.DMA,
            pltpu.SemaphoreType.DMA,
        ],
    )
    def k(h_hbm, gidx_hbm, sidx_hbm, z_hbm, out_hbm,
          sva, gva, svb, gvb, rva, rvb, acc_sh, sem_g, sem_s):
        cid = lax.axis_index("c")
        tid = lax.axis_index("s")

        pltpu.sync_copy(z_hbm, acc_sh.at[pl.ds(tid * _STRIPE, _STRIPE)])
        plsc.subcore_barrier()

        def load_idx(c, sv, gv):
            r0 = tid * 160 + c * krows
            pltpu.sync_copy(sidx_hbm.at[pl.ds(r0, krows)], sv)
            pltpu.sync_copy(gidx_hbm.at[pl.ds(cid * _ROWS + r0, krows)], gv)

        def fire_gathers(gv, rv):
            return [
                pltpu.async_copy(
                    h_hbm.at[gv.at[j]], rv.at[pl.ds(j * 128, 128)], sem_g
                )
                for j in range(krows)
            ]

        def fire_scatters(sv, rv):
            return [
                pltpu.async_copy(
                    rv.at[pl.ds(j * 128, 128)], acc_sh.at[sv.at[j]],
                    sem_s, add=True,
                )
                for j in range(krows)
            ]

        load_idx(0, sva, gva)

        def pair(c2, carry):
            c = c2 * 2
            ga = fire_gathers(gva, rva)
            load_idx(c + 1, svb, gvb)
            for d in ga:
                d.wait()
            sa = fire_scatters(sva, rva)
            gb = fire_gathers(gvb, rvb)
            for d in sa:
                d.wait()

            @pl.when(c2 + 1 < npairs)
            def _():
                load_idx(c + 2, sva, gva)

            for d in gb:
                d.wait()
            sb = fire_scatters(svb, rvb)
            for d in sb:
                d.wait()
            return carry

        lax.fori_loop(0, npairs, pair, 0)
        plsc.subcore_barrier()
        pltpu.sync_copy(
            acc_sh.at[pl.ds(tid * _STRIPE, _STRIPE)],
            out_hbm.at[pl.ds(cid * _ACC_ROWS + tid * _STRIPE, _STRIPE)],
        )

    return k(h_stack, gidx, sidx, z_acc)


def _edge_agg(h_stack, gidx, sidx, z_acc, dh, krows):
    mesh = plsc.VectorSubcoreMesh(core_axis_name="c", subcore_axis_name="s")

    @functools.partial(
        pl.kernel,
        out_type=jax.ShapeDtypeStruct((2 * _ACC_ROWS, dh), jnp.float32),
        mesh=mesh,
        compiler_params=pltpu.CompilerParams(use_tc_tiling_on_sc=False),
        scratch_types=[
            pltpu.VMEM((krows, 128), jnp.int32),
            pltpu.VMEM((krows, 128), jnp.int32),
            pltpu.VMEM((krows * 128, dh), jnp.float32),
            pltpu.VMEM_SHARED((_ACC_ROWS, dh), jnp.float32),
            pltpu.SemaphoreType.DMA,
            pltpu.SemaphoreType.DMA,
        ],
    )
    def k(h_hbm, gidx_hbm, sidx_hbm, z_hbm,
          out_hbm, sidx_v, gidx_v, rows_v, acc_sh, sem, sem2):
        cid = lax.axis_index("c")
        tid = lax.axis_index("s")

        pltpu.sync_copy(z_hbm, acc_sh.at[pl.ds(tid * _STRIPE, _STRIPE)])
        plsc.subcore_barrier()

        def chunk(c, carry):
            r0 = tid * 160 + c * krows
            pltpu.sync_copy(sidx_hbm.at[pl.ds(r0, krows)], sidx_v)
            pltpu.sync_copy(gidx_hbm.at[pl.ds(cid * _ROWS + r0, krows)], gidx_v)
            gathers = [
                pltpu.async_copy(
                    h_hbm.at[gidx_v.at[j]],
                    rows_v.at[pl.ds(j * 128, 128)],
                    sem,
                )
                for j in range(krows)
            ]
            scatters = []
            for j in range(krows):
                gathers[j].wait()
                scatters.append(pltpu.async_copy(
                    rows_v.at[pl.ds(j * 128, 128)],
                    acc_sh.at[sidx_v.at[j]],
                    sem2,
                    add=True,
                ))
            for d in scatters:
                d.wait()
            return carry

        lax.fori_loop(0, 160 // krows, chunk, 0)
        plsc.subcore_barrier()

        pltpu.sync_copy(
            acc_sh.at[pl.ds(tid * _STRIPE, _STRIPE)],
            out_hbm.at[pl.ds(cid * _ACC_ROWS + tid * _STRIPE, _STRIPE)],
        )

    return k(h_stack, gidx, sidx, z_acc)


def _edge_counts(sidx, z_cnt, ones):
    mesh = plsc.VectorSubcoreMesh(core_axis_name="c", subcore_axis_name="s")
    half = _ROWS // 2

    @functools.partial(
        pl.kernel,
        out_type=jax.ShapeDtypeStruct((2 * _ACC_ROWS, 16), jnp.float32),
        mesh=mesh,
        compiler_params=pltpu.CompilerParams(use_tc_tiling_on_sc=False),
        scratch_types=[
            pltpu.VMEM((8, 128), jnp.int32),
            pltpu.VMEM((128, 16), jnp.float32),
            pltpu.VMEM_SHARED((_ACC_ROWS, 16), jnp.float32),
        ],
    )
    def k(sidx_hbm, z_hbm, ones_hbm, out_hbm, sidx_v, ones_v, cnt_sh):
        cid = lax.axis_index("c")
        tid = lax.axis_index("s")
        pltpu.sync_copy(z_hbm, cnt_sh.at[pl.ds(tid * _STRIPE, _STRIPE)])
        pltpu.sync_copy(ones_hbm, ones_v)
        plsc.subcore_barrier()

        def chunk(c, carry):
            r0 = cid * half + tid * (half // 16) + c * 8
            pltpu.sync_copy(sidx_hbm.at[pl.ds(r0, 8)], sidx_v)
            for j in range(8):
                pltpu.sync_copy(ones_v, cnt_sh.at[sidx_v.at[j]], add=True)
            return carry

        lax.fori_loop(0, half // 16 // 8, chunk, 0)
        plsc.subcore_barrier()
        pltpu.sync_copy(
            cnt_sh.at[pl.ds(tid * _STRIPE, _STRIPE)],
            out_hbm.at[pl.ds(cid * _ACC_ROWS + tid * _STRIPE, _STRIPE)],
        )

    return k(sidx, z_cnt, ones)


def _dense_layer(h, a0, a1, a2, cnt8, root, w, b2d, emit_stack=False):
    din = h.shape[1]
    br = 2000
    grid = (_N // br,)

    def body(h_ref, a0_ref, a1_ref, a2_ref, c_ref, root_ref, w_ref, b_ref,
             o_ref, *o2):
        acc = jnp.dot(h_ref[...], root_ref[...],
                      preferred_element_type=jnp.float32,
                      precision=lax.Precision.HIGHEST)
        for r, aref in enumerate((a0_ref, a1_ref, a2_ref)):
            recip = 1.0 / jnp.maximum(c_ref[:, r:r + 1], 1.0)
            acc = acc + jnp.dot(aref[...].astype(jnp.float32) * recip, w_ref[r],
                                preferred_element_type=jnp.float32,
                                precision=lax.Precision.HIGHEST)
        acc = jnp.maximum(acc + b_ref[...], 0.0)
        o_ref[...] = acc
        if emit_stack:
            o2[0][0, :, :] = acc[:, :64].astype(jnp.bfloat16)
            o2[0][1, :, :] = acc[:, 64:].astype(jnp.bfloat16)

    node_spec = pl.BlockSpec((br, din), lambda i: (i, 0))
    out_specs = [pl.BlockSpec((br, 128), lambda i: (i, 0))]
    out_shape = [jax.ShapeDtypeStruct((_N, 128), jnp.float32)]
    if emit_stack:
        out_specs.append(pl.BlockSpec((2, br, 64), lambda i: (0, i, 0)))
        out_shape.append(jax.ShapeDtypeStruct((2, _N, 64), jnp.bfloat16))
    res = pl.pallas_call(
        body,
        grid=grid,
        in_specs=[
            node_spec, node_spec, node_spec, node_spec,
            pl.BlockSpec((br, 8), lambda i: (i, 0)),
            pl.BlockSpec((din, 128), lambda i: (0, 0)),
            pl.BlockSpec((3, din, 128), lambda i: (0, 0, 0)),
            pl.BlockSpec((1, 128), lambda i: (0, 0)),
        ],
        out_specs=out_specs,
        out_shape=out_shape,
    )(h, a0, a1, a2, cnt8, root, w, b2d)
    return res if emit_stack else res[0]


def _pool_classify(h2, batch3d, wc, bc2d):
    br = 2000
    nsteps = _N // br

    def body(h_ref, b_ref, wc_ref, bc_ref, o_ref, sums, cnts):
        i = pl.program_id(0)

        @pl.when(i == 0)
        def _():
            sums[...] = jnp.zeros((_NUM_GRAPHS, 128), jnp.float32)
            cnts[...] = jnp.zeros((_NUM_GRAPHS, 8), jnp.float32)

        bb = b_ref[0, 0, :]
        onehot = (bb[:, None] == lax.broadcasted_iota(
            jnp.int32, (br, _NUM_GRAPHS), 1)).astype(jnp.float32)
        sums[...] += lax.dot_general(
            onehot, h_ref[...], (((0,), (0,)), ((), ())),
            preferred_element_type=jnp.float32,
            precision=lax.Precision.HIGHEST)
        cnts[:, 0:1] += lax.dot_general(
            onehot, jnp.ones((br, 1), jnp.float32), (((0,), (0,)), ((), ())),
            preferred_element_type=jnp.float32,
            precision=lax.Precision.HIGHEST)

        @pl.when(i == nsteps - 1)
        def _():
            recip = 1.0 / jnp.maximum(cnts[:, 0:1], 1.0)
            pooled = sums[...] * recip
            o_ref[...] = jnp.dot(pooled, wc_ref[...],
                                 preferred_element_type=jnp.float32,
                                 precision=lax.Precision.HIGHEST) + bc_ref[...]

    return pl.pallas_call(
        body,
        grid=(nsteps,),
        in_specs=[
            pl.BlockSpec((br, 128), lambda i: (i, 0)),
            pl.BlockSpec((1, 1, br), lambda i: (i, 0, 0)),
            pl.BlockSpec((128, 16), lambda i: (0, 0)),
            pl.BlockSpec((1, 16), lambda i: (0, 0)),
        ],
        out_specs=pl.BlockSpec((_NUM_GRAPHS, 16), lambda i: (0, 0)),
        out_shape=jax.ShapeDtypeStruct((_NUM_GRAPHS, 16), jnp.float32),
        scratch_shapes=[
            pltpu.VMEM((_NUM_GRAPHS, 128), jnp.float32),
            pltpu.VMEM((_NUM_GRAPHS, 8), jnp.float32),
        ],
    )(h2, batch3d, wc, bc2d)


def kernel(x, edge_index, edge_type, batch, shape_emb, color_emb, pos_emb,
           W1, root1, b1, W2, root2, b2, Wc, bc):
    tbl = jnp.concatenate([shape_emb, color_emb, pos_emb], axis=0)
    idx_flat = jnp.stack(
        [x[:, 0], x[:, 1] + 16, jnp.clip(x[:, 2], 0, 24) + 32], axis=1
    ).reshape(-1)
    idx_pad = jnp.concatenate(
        [idx_flat, jnp.zeros((_EMB_PAD - 3 * _N,), jnp.int32)]
    ).reshape(_EMB_PAD // 128, 128)
    e = _emb_gather(tbl, idx_pad)
    h0 = e[: 3 * _N].reshape(_N, 96)

    src = edge_index[0]
    dst = edge_index[1]
    pad = _EP - _E
    srcp = jnp.concatenate([src, jnp.zeros((pad,), jnp.int32)])
    sidx = jnp.concatenate(
        [edge_type * _N + dst, jnp.full((pad,), 3 * _N, jnp.int32)]
    ).reshape(_ROWS, 128)
    gidx = jnp.concatenate([srcp, srcp + _N]).reshape(2 * _ROWS, 128)

    z48 = jnp.zeros((_STRIPE, 48), jnp.float32)
    z16 = jnp.zeros((_STRIPE, 16), jnp.float32)
    ones = jnp.ones((128, 16), jnp.float32)

    cnt_raw = _edge_counts(sidx, z16, ones)
    cnt_sum = cnt_raw[:_ACC_ROWS] + cnt_raw[_ACC_ROWS:]
    cnt = cnt_sum[: 3 * _N, 0].reshape(3, _N)
    cnt8 = jnp.pad(cnt.T, ((0, 0), (0, 5)))

    h0_stack = h0.reshape(_N, 2, 48).transpose(1, 0, 2).reshape(2 * _N, 48)
    a1_raw = _edge_agg(h0_stack, gidx, sidx, z48, 48, 4)
    a1 = (a1_raw.reshape(2, _ACC_ROWS, 48)[:, : 3 * _N]
          .reshape(2, 3, _N, 48).transpose(1, 2, 0, 3).reshape(3, _N, 96))
    h1, h1_stack = _dense_layer(h0, a1[0], a1[1], a1[2], cnt8, root1, W1,
                                b1.reshape(1, 128), emit_stack=True)

    zb = jnp.zeros((_STRIPE, 64), jnp.bfloat16)
    a2_raw = _edge_agg_pipe(h1_stack.reshape(2 * _N, 64), gidx, sidx, zb,
                            64, 8, jnp.bfloat16)
    a2 = (a2_raw.reshape(2, _ACC_ROWS, 64)[:, : 3 * _N]
          .reshape(2, 3, _N, 64).transpose(1, 2, 0, 3).reshape(3, _N, 128))
    h2 = _dense_layer(h1, a2[0], a2[1], a2[2], cnt8, root2, W2,
                      b2.reshape(1, 128))

    return _pool_classify(h2, batch.reshape(_N // 2000, 1, 2000), Wc,
                          bc.reshape(1, 16))


# speedup vs baseline: 2.0256x; 2.0256x over previous
import functools

import jax
import jax.numpy as jnp
from jax import lax
from jax.experimental import pallas as pl
from jax.experimental.pallas import tpu as pltpu
from jax.experimental.pallas import tpu_sc as plsc

_N = 10000
_E = 320000
_NUM_GRAPHS = 256
_NUM_REL = 3
_EP = 327680
_ROWS = _EP // 128
_ACC_ROWS = 30080
_STRIPE = _ACC_ROWS // 16
_EMB_PAD = 32768


def _emb_gather(tbl, idx2d):
    mesh = plsc.VectorSubcoreMesh(core_axis_name="c", subcore_axis_name="s")

    @functools.partial(
        pl.kernel,
        out_type=jax.ShapeDtypeStruct((_EMB_PAD, 32), jnp.float32),
        mesh=mesh,
        compiler_params=pltpu.CompilerParams(use_tc_tiling_on_sc=False),
        scratch_types=[
            pltpu.VMEM((8, 128), jnp.int32),
            pltpu.VMEM((1024, 32), jnp.float32),
            pltpu.SemaphoreType.DMA,
        ],
    )
    def k(tbl_hbm, idx_hbm, out_hbm, idx_v, rows_v, sem):
        cid = lax.axis_index("c")
        sid = lax.axis_index("s")
        wid = sid * 2 + cid
        pltpu.sync_copy(idx_hbm.at[pl.ds(wid * 8, 8)], idx_v)
        gathers = [
            pltpu.async_copy(
                tbl_hbm.at[idx_v.at[j]], rows_v.at[pl.ds(j * 128, 128)], sem
            )
            for j in range(8)
        ]
        for d in gathers:
            d.wait()
        pltpu.sync_copy(rows_v, out_hbm.at[pl.ds(wid * 1024, 1024)])

    return k(tbl, idx2d)


def _edge_agg_pipe(h_stack, gidx, sidx, z_acc, dh, krows, dtype=jnp.float32):
    mesh = plsc.VectorSubcoreMesh(core_axis_name="c", subcore_axis_name="s")
    npairs = 160 // krows // 2

    @functools.partial(
        pl.kernel,
        out_type=jax.ShapeDtypeStruct((2 * _ACC_ROWS, dh), dtype),
        mesh=mesh,
        compiler_params=pltpu.CompilerParams(use_tc_tiling_on_sc=False),
        scratch_types=[
            pltpu.VMEM((krows, 128), jnp.int32),
            pltpu.VMEM((krows, 128), jnp.int32),
            pltpu.VMEM((krows, 128), jnp.int32),
            pltpu.VMEM((krows, 128), jnp.int32),
            pltpu.VMEM((krows * 128, dh), dtype),
            pltpu.VMEM((krows * 128, dh), dtype),
            pltpu.VMEM_SHARED((_ACC_ROWS, dh), dtype),
            pltpu.SemaphoreType.DMA,
            pltpu.SemaphoreType.DMA,
        ],
    )
    def k(h_hbm, gidx_hbm, sidx_hbm, z_hbm, out_hbm,
          sva, gva, svb, gvb, rva, rvb, acc_sh, sem_g, sem_s):
        cid = lax.axis_index("c")
        tid = lax.axis_index("s")

        pltpu.sync_copy(z_hbm, acc_sh.at[pl.ds(tid * _STRIPE, _STRIPE)])
        plsc.subcore_barrier()

        def load_idx(c, sv, gv):
            r0 = tid * 160 + c * krows
            pltpu.sync_copy(sidx_hbm.at[pl.ds(r0, krows)], sv)
            pltpu.sync_copy(gidx_hbm.at[pl.ds(cid * _ROWS + r0, krows)], gv)

        def fire_gathers(gv, rv):
            return [
                pltpu.async_copy(
                    h_hbm.at[gv.at[j]], rv.at[pl.ds(j * 128, 128)], sem_g
                )
                for j in range(krows)
            ]

        def fire_scatters(sv, rv):
            return [
                pltpu.async_copy(
                    rv.at[pl.ds(j * 128, 128)], acc_sh.at[sv.at[j]],
                    sem_s, add=True,
                )
                for j in range(krows)
            ]

        load_idx(0, sva, gva)

        def pair(c2, carry):
            c = c2 * 2
            ga = fire_gathers(gva, rva)
            load_idx(c + 1, svb, gvb)
            for d in ga:
                d.wait()
            sa = fire_scatters(sva, rva)
            gb = fire_gathers(gvb, rvb)
            for d in sa:
                d.wait()

            @pl.when(c2 + 1 < npairs)
            def _():
                load_idx(c + 2, sva, gva)

            for d in gb:
                d.wait()
            sb = fire_scatters(svb, rvb)
            for d in sb:
                d.wait()
            return carry

        lax.fori_loop(0, npairs, pair, 0)
        plsc.subcore_barrier()
        pltpu.sync_copy(
            acc_sh.at[pl.ds(tid * _STRIPE, _STRIPE)],
            out_hbm.at[pl.ds(cid * _ACC_ROWS + tid * _STRIPE, _STRIPE)],
        )

    return k(h_stack, gidx, sidx, z_acc)


def _edge_agg(h_stack, gidx, sidx, z_acc, dh, krows):
    mesh = plsc.VectorSubcoreMesh(core_axis_name="c", subcore_axis_name="s")

    @functools.partial(
        pl.kernel,
        out_type=jax.ShapeDtypeStruct((2 * _ACC_ROWS, dh), jnp.float32),
        mesh=mesh,
        compiler_params=pltpu.CompilerParams(use_tc_tiling_on_sc=False),
        scratch_types=[
            pltpu.VMEM((krows, 128), jnp.int32),
            pltpu.VMEM((krows, 128), jnp.int32),
            pltpu.VMEM((krows * 128, dh), jnp.float32),
            pltpu.VMEM_SHARED((_ACC_ROWS, dh), jnp.float32),
            pltpu.SemaphoreType.DMA,
            pltpu.SemaphoreType.DMA,
        ],
    )
    def k(h_hbm, gidx_hbm, sidx_hbm, z_hbm,
          out_hbm, sidx_v, gidx_v, rows_v, acc_sh, sem, sem2):
        cid = lax.axis_index("c")
        tid = lax.axis_index("s")

        pltpu.sync_copy(z_hbm, acc_sh.at[pl.ds(tid * _STRIPE, _STRIPE)])
        plsc.subcore_barrier()

        def chunk(c, carry):
            r0 = tid * 160 + c * krows
            pltpu.sync_copy(sidx_hbm.at[pl.ds(r0, krows)], sidx_v)
            pltpu.sync_copy(gidx_hbm.at[pl.ds(cid * _ROWS + r0, krows)], gidx_v)
            gathers = [
                pltpu.async_copy(
                    h_hbm.at[gidx_v.at[j]],
                    rows_v.at[pl.ds(j * 128, 128)],
                    sem,
                )
                for j in range(krows)
            ]
            scatters = []
            for j in range(krows):
                gathers[j].wait()
                scatters.append(pltpu.async_copy(
                    rows_v.at[pl.ds(j * 128, 128)],
                    acc_sh.at[sidx_v.at[j]],
                    sem2,
                    add=True,
                ))
            for d in scatters:
                d.wait()
            return carry

        lax.fori_loop(0, 160 // krows, chunk, 0)
        plsc.subcore_barrier()

        pltpu.sync_copy(
            acc_sh.at[pl.ds(tid * _STRIPE, _STRIPE)],
            out_hbm.at[pl.ds(cid * _ACC_ROWS + tid * _STRIPE, _STRIPE)],
        )

    return k(h_stack, gidx, sidx, z_acc)


def _edge_counts(sidx, z_cnt, ones):
    mesh = plsc.VectorSubcoreMesh(core_axis_name="c", subcore_axis_name="s")
    half = _ROWS // 2

    @functools.partial(
        pl.kernel,
        out_type=jax.ShapeDtypeStruct((2 * _ACC_ROWS, 16), jnp.float32),
        mesh=mesh,
        compiler_params=pltpu.CompilerParams(use_tc_tiling_on_sc=False),
        scratch_types=[
            pltpu.VMEM((8, 128), jnp.int32),
            pltpu.VMEM((128, 16), jnp.float32),
            pltpu.VMEM_SHARED((_ACC_ROWS, 16), jnp.float32),
        ],
    )
    def k(sidx_hbm, z_hbm, ones_hbm, out_hbm, sidx_v, ones_v, cnt_sh):
        cid = lax.axis_index("c")
        tid = lax.axis_index("s")
        pltpu.sync_copy(z_hbm, cnt_sh.at[pl.ds(tid * _STRIPE, _STRIPE)])
        pltpu.sync_copy(ones_hbm, ones_v)
        plsc.subcore_barrier()

        def chunk(c, carry):
            r0 = cid * half + tid * (half // 16) + c * 8
            pltpu.sync_copy(sidx_hbm.at[pl.ds(r0, 8)], sidx_v)
            for j in range(8):
                pltpu.sync_copy(ones_v, cnt_sh.at[sidx_v.at[j]], add=True)
            return carry

        lax.fori_loop(0, half // 16 // 8, chunk, 0)
        plsc.subcore_barrier()
        pltpu.sync_copy(
            cnt_sh.at[pl.ds(tid * _STRIPE, _STRIPE)],
            out_hbm.at[pl.ds(cid * _ACC_ROWS + tid * _STRIPE, _STRIPE)],
        )

    return k(sidx, z_cnt, ones)


def _dense_layer(h, a0, a1, a2, cnt8, root, w, b2d, emit_stack=False):
    din = h.shape[1]
    br = 2000
    grid = (_N // br,)

    def body(h_ref, a0_ref, a1_ref, a2_ref, c_ref, root_ref, w_ref, b_ref,
             o_ref, *o2):
        acc = jnp.dot(h_ref[...], root_ref[...],
                      preferred_element_type=jnp.float32,
                      precision=lax.Precision.HIGHEST)
        for r, aref in enumerate((a0_ref, a1_ref, a2_ref)):
            recip = 1.0 / jnp.maximum(c_ref[:, r:r + 1], 1.0)
            acc = acc + jnp.dot(aref[...].astype(jnp.float32) * recip, w_ref[r],
                                preferred_element_type=jnp.float32,
                                precision=lax.Precision.HIGHEST)
        acc = jnp.maximum(acc + b_ref[...], 0.0)
        o_ref[...] = acc
        if emit_stack:
            for q in range(4):
                o2[0][q, :, :] = acc[:, 32 * q:32 * q + 32]

    node_spec = pl.BlockSpec((br, din), lambda i: (i, 0))
    out_specs = [pl.BlockSpec((br, 128), lambda i: (i, 0))]
    out_shape = [jax.ShapeDtypeStruct((_N, 128), jnp.float32)]
    if emit_stack:
        out_specs.append(pl.BlockSpec((4, br, 32), lambda i: (0, i, 0)))
        out_shape.append(jax.ShapeDtypeStruct((4, _N, 32), jnp.float32))
    res = pl.pallas_call(
        body,
        grid=grid,
        in_specs=[
            node_spec, node_spec, node_spec, node_spec,
            pl.BlockSpec((br, 8), lambda i: (i, 0)),
            pl.BlockSpec((din, 128), lambda i: (0, 0)),
            pl.BlockSpec((3, din, 128), lambda i: (0, 0, 0)),
            pl.BlockSpec((1, 128), lambda i: (0, 0)),
        ],
        out_specs=out_specs,
        out_shape=out_shape,
    )(h, a0, a1, a2, cnt8, root, w, b2d)
    return res if emit_stack else res[0]


def _pool_classify(h2, batch3d, wc, bc2d):
    br = 2000
    nsteps = _N // br

    def body(h_ref, b_ref, wc_ref, bc_ref, o_ref, sums, cnts):
        i = pl.program_id(0)

        @pl.when(i == 0)
        def _():
            sums[...] = jnp.zeros((_NUM_GRAPHS, 128), jnp.float32)
            cnts[...] = jnp.zeros((_NUM_GRAPHS, 8), jnp.float32)

        bb = b_ref[0, 0, :]
        onehot = (bb[:, None] == lax.broadcasted_iota(
            jnp.int32, (br, _NUM_GRAPHS), 1)).astype(jnp.float32)
        sums[...] += lax.dot_general(
            onehot, h_ref[...], (((0,), (0,)), ((), ())),
            preferred_element_type=jnp.float32,
            precision=lax.Precision.HIGHEST)
        cnts[:, 0:1] += lax.dot_general(
            onehot, jnp.ones((br, 1), jnp.float32), (((0,), (0,)), ((), ())),
            preferred_element_type=jnp.float32,
            precision=lax.Precision.HIGHEST)

        @pl.when(i == nsteps - 1)
        def _():
            recip = 1.0 / jnp.maximum(cnts[:, 0:1], 1.0)
            pooled = sums[...] * recip
            o_ref[...] = jnp.dot(pooled, wc_ref[...],
                                 preferred_element_type=jnp.float32,
                                 precision=lax.Precision.HIGHEST) + bc_ref[...]

    return pl.pallas_call(
        body,
        grid=(nsteps,),
        in_specs=[
            pl.BlockSpec((br, 128), lambda i: (i, 0)),
            pl.BlockSpec((1, 1, br), lambda i: (i, 0, 0)),
            pl.BlockSpec((128, 16), lambda i: (0, 0)),
            pl.BlockSpec((1, 16), lambda i: (0, 0)),
        ],
        out_specs=pl.BlockSpec((_NUM_GRAPHS, 16), lambda i: (0, 0)),
        out_shape=jax.ShapeDtypeStruct((_NUM_GRAPHS, 16), jnp.float32),
        scratch_shapes=[
            pltpu.VMEM((_NUM_GRAPHS, 128), jnp.float32),
            pltpu.VMEM((_NUM_GRAPHS, 8), jnp.float32),
        ],
    )(h2, batch3d, wc, bc2d)


def kernel(x, edge_index, edge_type, batch, shape_emb, color_emb, pos_emb,
           W1, root1, b1, W2, root2, b2, Wc, bc):
    tbl = jnp.concatenate([shape_emb, color_emb, pos_emb], axis=0)
    idx_flat = jnp.stack(
        [x[:, 0], x[:, 1] + 16, jnp.clip(x[:, 2], 0, 24) + 32], axis=1
    ).reshape(-1)
    idx_pad = jnp.concatenate(
        [idx_flat, jnp.zeros((_EMB_PAD - 3 * _N,), jnp.int32)]
    ).reshape(_EMB_PAD // 128, 128)
    e = _emb_gather(tbl, idx_pad)
    h0 = e[: 3 * _N].reshape(_N, 96)

    src = edge_index[0]
    dst = edge_index[1]
    pad = _EP - _E
    srcp = jnp.concatenate([src, jnp.zeros((pad,), jnp.int32)])
    sidx = jnp.concatenate(
        [edge_type * _N + dst, jnp.full((pad,), 3 * _N, jnp.int32)]
    ).reshape(_ROWS, 128)
    gidx = jnp.concatenate([srcp, srcp + _N]).reshape(2 * _ROWS, 128)

    z48 = jnp.zeros((_STRIPE, 48), jnp.float32)
    z16 = jnp.zeros((_STRIPE, 16), jnp.float32)
    ones = jnp.ones((128, 16), jnp.float32)

    cnt_raw = _edge_counts(sidx, z16, ones)
    cnt_sum = cnt_raw[:_ACC_ROWS] + cnt_raw[_ACC_ROWS:]
    cnt = cnt_sum[: 3 * _N, 0].reshape(3, _N)
    cnt8 = jnp.pad(cnt.T, ((0, 0), (0, 5)))

    h0_stack = h0.reshape(_N, 2, 48).transpose(1, 0, 2).reshape(2 * _N, 48)
    a1_raw = _edge_agg(h0_stack, gidx, sidx, z48, 48, 4)
    a1 = (a1_raw.reshape(2, _ACC_ROWS, 48)[:, : 3 * _N]
          .reshape(2, 3, _N, 48).transpose(1, 2, 0, 3).reshape(3, _N, 96))
    h1, h1_stack = _dense_layer(h0, a1[0], a1[1], a1[2], cnt8, root1, W1,
                                b1.reshape(1, 128), emit_stack=True)

    z32 = jnp.zeros((_STRIPE, 32), jnp.float32)
    stack4 = h1_stack.reshape(4 * _N, 32)
    a2a_raw = _edge_agg_pipe(stack4[: 2 * _N], gidx, sidx, z32, 32, 8)
    a2b_raw = _edge_agg_pipe(stack4[2 * _N:], gidx, sidx, z32, 32, 8)
    a2a = (a2a_raw.reshape(2, _ACC_ROWS, 32)[:, : 3 * _N]
           .reshape(2, 3, _N, 32).transpose(1, 2, 0, 3).reshape(3, _N, 64))
    a2b = (a2b_raw.reshape(2, _ACC_ROWS, 32)[:, : 3 * _N]
           .reshape(2, 3, _N, 32).transpose(1, 2, 0, 3).reshape(3, _N, 64))
    a2 = jnp.concatenate([a2a, a2b], axis=-1)
    h2 = _dense_layer(h1, a2[0], a2[1], a2[2], cnt8, root2, W2,
                      b2.reshape(1, 128))

    return _pool_classify(h2, batch.reshape(_N // 2000, 1, 2000), Wc,
                          bc.reshape(1, 16))


# speedup vs baseline: 2.0477x; 1.0109x over previous
import functools

import jax
import jax.numpy as jnp
from jax import lax
from jax.experimental import pallas as pl
from jax.experimental.pallas import tpu as pltpu
from jax.experimental.pallas import tpu_sc as plsc

_N = 10000
_E = 320000
_NUM_GRAPHS = 256
_NUM_REL = 3
_EP = 327680
_ROWS = _EP // 128
_ACC_ROWS = 30080
_STRIPE = _ACC_ROWS // 16
_EMB_PAD = 32768


def _emb_gather(tbl, idx2d):
    mesh = plsc.VectorSubcoreMesh(core_axis_name="c", subcore_axis_name="s")

    @functools.partial(
        pl.kernel,
        out_type=jax.ShapeDtypeStruct((_EMB_PAD, 32), jnp.float32),
        mesh=mesh,
        compiler_params=pltpu.CompilerParams(use_tc_tiling_on_sc=False),
        scratch_types=[
            pltpu.VMEM((8, 128), jnp.int32),
            pltpu.VMEM((1024, 32), jnp.float32),
            pltpu.SemaphoreType.DMA,
        ],
    )
    def k(tbl_hbm, idx_hbm, out_hbm, idx_v, rows_v, sem):
        cid = lax.axis_index("c")
        sid = lax.axis_index("s")
        wid = sid * 2 + cid
        pltpu.sync_copy(idx_hbm.at[pl.ds(wid * 8, 8)], idx_v)
        gathers = [
            pltpu.async_copy(
                tbl_hbm.at[idx_v.at[j]], rows_v.at[pl.ds(j * 128, 128)], sem
            )
            for j in range(8)
        ]
        for d in gathers:
            d.wait()
        pltpu.sync_copy(rows_v, out_hbm.at[pl.ds(wid * 1024, 1024)])

    return k(tbl, idx2d)


def _edge_agg_pipe(h_stack, gidx, sidx, z_acc, dh, krows, dtype=jnp.float32):
    mesh = plsc.VectorSubcoreMesh(core_axis_name="c", subcore_axis_name="s")
    npairs = 160 // krows // 2

    @functools.partial(
        pl.kernel,
        out_type=jax.ShapeDtypeStruct((2 * _ACC_ROWS, dh), dtype),
        mesh=mesh,
        compiler_params=pltpu.CompilerParams(use_tc_tiling_on_sc=False),
        scratch_types=[
            pltpu.VMEM((krows, 128), jnp.int32),
            pltpu.VMEM((krows, 128), jnp.int32),
            pltpu.VMEM((krows, 128), jnp.int32),
            pltpu.VMEM((krows, 128), jnp.int32),
            pltpu.VMEM((krows * 128, dh), dtype),
            pltpu.VMEM((krows * 128, dh), dtype),
            pltpu.VMEM_SHARED((_ACC_ROWS, dh), dtype),
            pltpu.SemaphoreType.DMA,
            pltpu.SemaphoreType.DMA,
        ],
    )
    def k(h_hbm, gidx_hbm, sidx_hbm, z_hbm, out_hbm,
          sva, gva, svb, gvb, rva, rvb, acc_sh, sem_g, sem_s):
        cid = lax.axis_index("c")
        tid = lax.axis_index("s")

        pltpu.sync_copy(z_hbm, acc_sh.at[pl.ds(tid * _STRIPE, _STRIPE)])
        plsc.subcore_barrier()

        def load_idx(c, sv, gv):
            r0 = tid * 160 + c * krows
            pltpu.sync_copy(sidx_hbm.at[pl.ds(r0, krows)], sv)
            pltpu.sync_copy(gidx_hbm.at[pl.ds(cid * _ROWS + r0, krows)], gv)

        def fire_gathers(gv, rv):
            return [
                pltpu.async_copy(
                    h_hbm.at[gv.at[j]], rv.at[pl.ds(j * 128, 128)], sem_g
                )
                for j in range(krows)
            ]

        def fire_scatters(sv, rv):
            return [
                pltpu.async_copy(
                    rv.at[pl.ds(j * 128, 128)], acc_sh.at[sv.at[j]],
                    sem_s, add=True,
                )
                for j in range(krows)
            ]

        load_idx(0, sva, gva)

        def pair(c2, carry):
            c = c2 * 2
            ga = fire_gathers(gva, rva)
            load_idx(c + 1, svb, gvb)
            for d in ga:
                d.wait()
            sa = fire_scatters(sva, rva)
            gb = fire_gathers(gvb, rvb)
            for d in sa:
                d.wait()

            @pl.when(c2 + 1 < npairs)
            def _():
                load_idx(c + 2, sva, gva)

            for d in gb:
                d.wait()
            sb = fire_scatters(svb, rvb)
            for d in sb:
                d.wait()
            return carry

        lax.fori_loop(0, npairs, pair, 0)
        plsc.subcore_barrier()
        pltpu.sync_copy(
            acc_sh.at[pl.ds(tid * _STRIPE, _STRIPE)],
            out_hbm.at[pl.ds(cid * _ACC_ROWS + tid * _STRIPE, _STRIPE)],
        )

    return k(h_stack, gidx, sidx, z_acc)


def _edge_agg(h_stack, gidx, sidx, z_acc, dh, krows):
    mesh = plsc.VectorSubcoreMesh(core_axis_name="c", subcore_axis_name="s")

    @functools.partial(
        pl.kernel,
        out_type=jax.ShapeDtypeStruct((2 * _ACC_ROWS, dh), jnp.float32),
        mesh=mesh,
        compiler_params=pltpu.CompilerParams(use_tc_tiling_on_sc=False),
        scratch_types=[
            pltpu.VMEM((krows, 128), jnp.int32),
            pltpu.VMEM((krows, 128), jnp.int32),
            pltpu.VMEM((krows * 128, dh), jnp.float32),
            pltpu.VMEM_SHARED((_ACC_ROWS, dh), jnp.float32),
            pltpu.SemaphoreType.DMA,
            pltpu.SemaphoreType.DMA,
        ],
    )
    def k(h_hbm, gidx_hbm, sidx_hbm, z_hbm,
          out_hbm, sidx_v, gidx_v, rows_v, acc_sh, sem, sem2):
        cid = lax.axis_index("c")
        tid = lax.axis_index("s")

        pltpu.sync_copy(z_hbm, acc_sh.at[pl.ds(tid * _STRIPE, _STRIPE)])
        plsc.subcore_barrier()

        def chunk(c, carry):
            r0 = tid * 160 + c * krows
            pltpu.sync_copy(sidx_hbm.at[pl.ds(r0, krows)], sidx_v)
            pltpu.sync_copy(gidx_hbm.at[pl.ds(cid * _ROWS + r0, krows)], gidx_v)
            gathers = [
                pltpu.async_copy(
                    h_hbm.at[gidx_v.at[j]],
                    rows_v.at[pl.ds(j * 128, 128)],
                    sem,
                )
                for j in range(krows)
            ]
            scatters = []
            for j in range(krows):
                gathers[j].wait()
                scatters.append(pltpu.async_copy(
                    rows_v.at[pl.ds(j * 128, 128)],
                    acc_sh.at[sidx_v.at[j]],
                    sem2,
                    add=True,
                ))
            for d in scatters:
                d.wait()
            return carry

        lax.fori_loop(0, 160 // krows, chunk, 0)
        plsc.subcore_barrier()

        pltpu.sync_copy(
            acc_sh.at[pl.ds(tid * _STRIPE, _STRIPE)],
            out_hbm.at[pl.ds(cid * _ACC_ROWS + tid * _STRIPE, _STRIPE)],
        )

    return k(h_stack, gidx, sidx, z_acc)


def _edge_counts(sidx, z_cnt, ones):
    mesh = plsc.VectorSubcoreMesh(core_axis_name="c", subcore_axis_name="s")
    half = _ROWS // 2

    @functools.partial(
        pl.kernel,
        out_type=jax.ShapeDtypeStruct((2 * _ACC_ROWS, 16), jnp.float32),
        mesh=mesh,
        compiler_params=pltpu.CompilerParams(use_tc_tiling_on_sc=False),
        scratch_types=[
            pltpu.VMEM((8, 128), jnp.int32),
            pltpu.VMEM((128, 16), jnp.float32),
            pltpu.VMEM_SHARED((_ACC_ROWS, 16), jnp.float32),
        ],
    )
    def k(sidx_hbm, z_hbm, ones_hbm, out_hbm, sidx_v, ones_v, cnt_sh):
        cid = lax.axis_index("c")
        tid = lax.axis_index("s")
        pltpu.sync_copy(z_hbm, cnt_sh.at[pl.ds(tid * _STRIPE, _STRIPE)])
        pltpu.sync_copy(ones_hbm, ones_v)
        plsc.subcore_barrier()

        def chunk(c, carry):
            r0 = cid * half + tid * (half // 16) + c * 8
            pltpu.sync_copy(sidx_hbm.at[pl.ds(r0, 8)], sidx_v)
            for j in range(8):
                pltpu.sync_copy(ones_v, cnt_sh.at[sidx_v.at[j]], add=True)
            return carry

        lax.fori_loop(0, half // 16 // 8, chunk, 0)
        plsc.subcore_barrier()
        pltpu.sync_copy(
            cnt_sh.at[pl.ds(tid * _STRIPE, _STRIPE)],
            out_hbm.at[pl.ds(cid * _ACC_ROWS + tid * _STRIPE, _STRIPE)],
        )

    return k(sidx, z_cnt, ones)


def _dense_layer(h, a0, a1, a2, cnt8, root, w, b2d, emit_stack=False):
    din = h.shape[1]
    br = 2000
    grid = (_N // br,)

    def body(h_ref, a0_ref, a1_ref, a2_ref, c_ref, root_ref, w_ref, b_ref,
             o_ref, *o2):
        acc = jnp.dot(h_ref[...], root_ref[...],
                      preferred_element_type=jnp.float32,
                      precision=lax.Precision.HIGHEST)
        for r, aref in enumerate((a0_ref, a1_ref, a2_ref)):
            recip = 1.0 / jnp.maximum(c_ref[:, r:r + 1], 1.0)
            acc = acc + jnp.dot(aref[...].astype(jnp.float32) * recip, w_ref[r],
                                preferred_element_type=jnp.float32,
                                precision=lax.Precision.HIGHEST)
        acc = jnp.maximum(acc + b_ref[...], 0.0)
        o_ref[...] = acc
        if emit_stack:
            for q in range(4):
                o2[0][q, :, :] = acc[:, 32 * q:32 * q + 32]

    node_spec = pl.BlockSpec((br, din), lambda i: (i, 0))
    out_specs = [pl.BlockSpec((br, 128), lambda i: (i, 0))]
    out_shape = [jax.ShapeDtypeStruct((_N, 128), jnp.float32)]
    if emit_stack:
        out_specs.append(pl.BlockSpec((4, br, 32), lambda i: (0, i, 0)))
        out_shape.append(jax.ShapeDtypeStruct((4, _N, 32), jnp.float32))
    res = pl.pallas_call(
        body,
        grid=grid,
        in_specs=[
            node_spec, node_spec, node_spec, node_spec,
            pl.BlockSpec((br, 8), lambda i: (i, 0)),
            pl.BlockSpec((din, 128), lambda i: (0, 0)),
            pl.BlockSpec((3, din, 128), lambda i: (0, 0, 0)),
            pl.BlockSpec((1, 128), lambda i: (0, 0)),
        ],
        out_specs=out_specs,
        out_shape=out_shape,
    )(h, a0, a1, a2, cnt8, root, w, b2d)
    return res if emit_stack else res[0]


def _pool_classify(h2, batch3d, wc, bc2d):
    br = 2000
    nsteps = _N // br

    def body(h_ref, b_ref, wc_ref, bc_ref, o_ref, sums, cnts):
        i = pl.program_id(0)

        @pl.when(i == 0)
        def _():
            sums[...] = jnp.zeros((_NUM_GRAPHS, 128), jnp.float32)
            cnts[...] = jnp.zeros((_NUM_GRAPHS, 8), jnp.float32)

        bb = b_ref[0, 0, :]
        onehot = (bb[:, None] == lax.broadcasted_iota(
            jnp.int32, (br, _NUM_GRAPHS), 1)).astype(jnp.float32)
        sums[...] += lax.dot_general(
            onehot, h_ref[...], (((0,), (0,)), ((), ())),
            preferred_element_type=jnp.float32,
            precision=lax.Precision.HIGHEST)
        cnts[:, 0:1] += lax.dot_general(
            onehot, jnp.ones((br, 1), jnp.float32), (((0,), (0,)), ((), ())),
            preferred_element_type=jnp.float32,
            precision=lax.Precision.HIGHEST)

        @pl.when(i == nsteps - 1)
        def _():
            recip = 1.0 / jnp.maximum(cnts[:, 0:1], 1.0)
            pooled = sums[...] * recip
            o_ref[...] = jnp.dot(pooled, wc_ref[...],
                                 preferred_element_type=jnp.float32,
                                 precision=lax.Precision.HIGHEST) + bc_ref[...]

    return pl.pallas_call(
        body,
        grid=(nsteps,),
        in_specs=[
            pl.BlockSpec((br, 128), lambda i: (i, 0)),
            pl.BlockSpec((1, 1, br), lambda i: (i, 0, 0)),
            pl.BlockSpec((128, 16), lambda i: (0, 0)),
            pl.BlockSpec((1, 16), lambda i: (0, 0)),
        ],
        out_specs=pl.BlockSpec((_NUM_GRAPHS, 16), lambda i: (0, 0)),
        out_shape=jax.ShapeDtypeStruct((_NUM_GRAPHS, 16), jnp.float32),
        scratch_shapes=[
            pltpu.VMEM((_NUM_GRAPHS, 128), jnp.float32),
            pltpu.VMEM((_NUM_GRAPHS, 8), jnp.float32),
        ],
    )(h2, batch3d, wc, bc2d)


def kernel(x, edge_index, edge_type, batch, shape_emb, color_emb, pos_emb,
           W1, root1, b1, W2, root2, b2, Wc, bc):
    tbl = jnp.concatenate([shape_emb, color_emb, pos_emb], axis=0)
    idx_flat = jnp.stack(
        [x[:, 0], x[:, 1] + 16, jnp.clip(x[:, 2], 0, 24) + 32], axis=1
    ).reshape(-1)
    idx_pad = jnp.concatenate(
        [idx_flat, jnp.zeros((_EMB_PAD - 3 * _N,), jnp.int32)]
    ).reshape(_EMB_PAD // 128, 128)
    e = _emb_gather(tbl, idx_pad)
    h0 = e[: 3 * _N].reshape(_N, 96)

    src = edge_index[0]
    dst = edge_index[1]
    pad = _EP - _E
    srcp = jnp.concatenate([src, jnp.zeros((pad,), jnp.int32)])
    sidx = jnp.concatenate(
        [edge_type * _N + dst, jnp.full((pad,), 3 * _N, jnp.int32)]
    ).reshape(_ROWS, 128)
    gidx = jnp.concatenate([srcp, srcp + _N]).reshape(2 * _ROWS, 128)

    z48 = jnp.zeros((_STRIPE, 48), jnp.float32)
    z16 = jnp.zeros((_STRIPE, 16), jnp.float32)
    ones = jnp.ones((128, 16), jnp.float32)

    cnt_raw = _edge_counts(sidx, z16, ones)
    cnt_sum = cnt_raw[:_ACC_ROWS] + cnt_raw[_ACC_ROWS:]
    cnt = cnt_sum[: 3 * _N, 0].reshape(3, _N)
    cnt8 = jnp.pad(cnt.T, ((0, 0), (0, 5)))

    h0_stack = h0.reshape(_N, 2, 48).transpose(1, 0, 2).reshape(2 * _N, 48)
    a1_raw = _edge_agg(h0_stack, gidx, sidx, z48, 48, 5)
    a1 = (a1_raw.reshape(2, _ACC_ROWS, 48)[:, : 3 * _N]
          .reshape(2, 3, _N, 48).transpose(1, 2, 0, 3).reshape(3, _N, 96))
    h1, h1_stack = _dense_layer(h0, a1[0], a1[1], a1[2], cnt8, root1, W1,
                                b1.reshape(1, 128), emit_stack=True)

    z32 = jnp.zeros((_STRIPE, 32), jnp.float32)
    stack4 = h1_stack.reshape(4 * _N, 32)
    a2a_raw = _edge_agg_pipe(stack4[: 2 * _N], gidx, sidx, z32, 32, 8)
    a2b_raw = _edge_agg_pipe(stack4[2 * _N:], gidx, sidx, z32, 32, 8)
    a2a = (a2a_raw.reshape(2, _ACC_ROWS, 32)[:, : 3 * _N]
           .reshape(2, 3, _N, 32).transpose(1, 2, 0, 3).reshape(3, _N, 64))
    a2b = (a2b_raw.reshape(2, _ACC_ROWS, 32)[:, : 3 * _N]
           .reshape(2, 3, _N, 32).transpose(1, 2, 0, 3).reshape(3, _N, 64))
    a2 = jnp.concatenate([a2a, a2b], axis=-1)
    h2 = _dense_layer(h1, a2[0], a2[1], a2[2], cnt8, root2, W2,
                      b2.reshape(1, 128))

    return _pool_classify(h2, batch.reshape(_N // 2000, 1, 2000), Wc,
                          bc.reshape(1, 16))


# speedup vs baseline: 2.0531x; 1.0026x over previous
import functools

import jax
import jax.numpy as jnp
from jax import lax
from jax.experimental import pallas as pl
from jax.experimental.pallas import tpu as pltpu
from jax.experimental.pallas import tpu_sc as plsc

_N = 10000
_E = 320000
_NUM_GRAPHS = 256
_NUM_REL = 3
_EP = 327680
_ROWS = _EP // 128
_ACC_ROWS = 30080
_STRIPE = _ACC_ROWS // 16
_EMB_PAD = 32768


def _emb_gather(tbl, idx2d):
    mesh = plsc.VectorSubcoreMesh(core_axis_name="c", subcore_axis_name="s")

    @functools.partial(
        pl.kernel,
        out_type=jax.ShapeDtypeStruct((_EMB_PAD, 32), jnp.float32),
        mesh=mesh,
        compiler_params=pltpu.CompilerParams(use_tc_tiling_on_sc=False),
        scratch_types=[
            pltpu.VMEM((8, 128), jnp.int32),
            pltpu.VMEM((1024, 32), jnp.float32),
            pltpu.SemaphoreType.DMA,
        ],
    )
    def k(tbl_hbm, idx_hbm, out_hbm, idx_v, rows_v, sem):
        cid = lax.axis_index("c")
        sid = lax.axis_index("s")
        wid = sid * 2 + cid
        pltpu.sync_copy(idx_hbm.at[pl.ds(wid * 8, 8)], idx_v)
        gathers = [
            pltpu.async_copy(
                tbl_hbm.at[idx_v.at[j]], rows_v.at[pl.ds(j * 128, 128)], sem
            )
            for j in range(8)
        ]
        for d in gathers:
            d.wait()
        pltpu.sync_copy(rows_v, out_hbm.at[pl.ds(wid * 1024, 1024)])

    return k(tbl, idx2d)


def _edge_agg_pipe(h_stack, gidx, sidx, z_acc, dh, krows, dtype=jnp.float32):
    mesh = plsc.VectorSubcoreMesh(core_axis_name="c", subcore_axis_name="s")
    npairs = 160 // krows // 2

    @functools.partial(
        pl.kernel,
        out_type=jax.ShapeDtypeStruct((2 * _ACC_ROWS, dh), dtype),
        mesh=mesh,
        compiler_params=pltpu.CompilerParams(use_tc_tiling_on_sc=False),
        scratch_types=[
            pltpu.VMEM((krows, 128), jnp.int32),
            pltpu.VMEM((krows, 128), jnp.int32),
            pltpu.VMEM((krows, 128), jnp.int32),
            pltpu.VMEM((krows, 128), jnp.int32),
            pltpu.VMEM((krows * 128, dh), dtype),
            pltpu.VMEM((krows * 128, dh), dtype),
            pltpu.VMEM_SHARED((_ACC_ROWS, dh), dtype),
            pltpu.SemaphoreType.DMA,
            pltpu.SemaphoreType.DMA,
        ],
    )
    def k(h_hbm, gidx_hbm, sidx_hbm, z_hbm, out_hbm,
          sva, gva, svb, gvb, rva, rvb, acc_sh, sem_g, sem_s):
        cid = lax.axis_index("c")
        tid = lax.axis_index("s")

        pltpu.sync_copy(z_hbm, acc_sh.at[pl.ds(tid * _STRIPE, _STRIPE)])
        plsc.subcore_barrier()

        def load_idx(c, sv, gv):
            r0 = tid * 160 + c * krows
            pltpu.sync_copy(sidx_hbm.at[pl.ds(r0, krows)], sv)
            pltpu.sync_copy(gidx_hbm.at[pl.ds(cid * _ROWS + r0, krows)], gv)

        def fire_gathers(gv, rv):
            return [
                pltpu.async_copy(
                    h_hbm.at[gv.at[j]], rv.at[pl.ds(j * 128, 128)], sem_g
                )
                for j in range(krows)
            ]

        def fire_scatters(sv, rv):
            return [
                pltpu.async_copy(
                    rv.at[pl.ds(j * 128, 128)], acc_sh.at[sv.at[j]],
                    sem_s, add=True,
                )
                for j in range(krows)
            ]

        load_idx(0, sva, gva)

        def pair(c2, carry):
            c = c2 * 2
            ga = fire_gathers(gva, rva)
            load_idx(c + 1, svb, gvb)
            for d in ga:
                d.wait()
            sa = fire_scatters(sva, rva)
            gb = fire_gathers(gvb, rvb)
            for d in sa:
                d.wait()

            @pl.when(c2 + 1 < npairs)
            def _():
                load_idx(c + 2, sva, gva)

            for d in gb:
                d.wait()
            sb = fire_scatters(svb, rvb)
            for d in sb:
                d.wait()
            return carry

        lax.fori_loop(0, npairs, pair, 0)
        plsc.subcore_barrier()
        pltpu.sync_copy(
            acc_sh.at[pl.ds(tid * _STRIPE, _STRIPE)],
            out_hbm.at[pl.ds(cid * _ACC_ROWS + tid * _STRIPE, _STRIPE)],
        )

    return k(h_stack, gidx, sidx, z_acc)


def _edge_agg(h_stack, gidx, sidx, z_acc, dh, krows):
    mesh = plsc.VectorSubcoreMesh(core_axis_name="c", subcore_axis_name="s")

    @functools.partial(
        pl.kernel,
        out_type=jax.ShapeDtypeStruct((2 * _ACC_ROWS, dh), jnp.float32),
        mesh=mesh,
        compiler_params=pltpu.CompilerParams(use_tc_tiling_on_sc=False),
        scratch_types=[
            pltpu.VMEM((krows, 128), jnp.int32),
            pltpu.VMEM((krows, 128), jnp.int32),
            pltpu.VMEM((krows * 128, dh), jnp.float32),
            pltpu.VMEM_SHARED((_ACC_ROWS, dh), jnp.float32),
            pltpu.SemaphoreType.DMA,
            pltpu.SemaphoreType.DMA,
        ],
    )
    def k(h_hbm, gidx_hbm, sidx_hbm, z_hbm,
          out_hbm, sidx_v, gidx_v, rows_v, acc_sh, sem, sem2):
        cid = lax.axis_index("c")
        tid = lax.axis_index("s")

        pltpu.sync_copy(z_hbm, acc_sh.at[pl.ds(tid * _STRIPE, _STRIPE)])
        plsc.subcore_barrier()

        def chunk(c, carry):
            r0 = tid * 160 + c * krows
            pltpu.sync_copy(sidx_hbm.at[pl.ds(r0, krows)], sidx_v)
            pltpu.sync_copy(gidx_hbm.at[pl.ds(cid * _ROWS + r0, krows)], gidx_v)
            gathers = [
                pltpu.async_copy(
                    h_hbm.at[gidx_v.at[j]],
                    rows_v.at[pl.ds(j * 128, 128)],
                    sem,
                )
                for j in range(krows)
            ]
            scatters = []
            for j in range(krows):
                gathers[j].wait()
                scatters.append(pltpu.async_copy(
                    rows_v.at[pl.ds(j * 128, 128)],
                    acc_sh.at[sidx_v.at[j]],
                    sem2,
                    add=True,
                ))
            for d in scatters:
                d.wait()
            return carry

        lax.fori_loop(0, 160 // krows, chunk, 0)
        plsc.subcore_barrier()

        pltpu.sync_copy(
            acc_sh.at[pl.ds(tid * _STRIPE, _STRIPE)],
            out_hbm.at[pl.ds(cid * _ACC_ROWS + tid * _STRIPE, _STRIPE)],
        )

    return k(h_stack, gidx, sidx, z_acc)


def _edge_counts(sidx, z_cnt, ones):
    mesh = plsc.VectorSubcoreMesh(core_axis_name="c", subcore_axis_name="s")
    half = _ROWS // 2

    @functools.partial(
        pl.kernel,
        out_type=jax.ShapeDtypeStruct((2 * _ACC_ROWS, 16), jnp.float32),
        mesh=mesh,
        compiler_params=pltpu.CompilerParams(use_tc_tiling_on_sc=False),
        scratch_types=[
            pltpu.VMEM((8, 128), jnp.int32),
            pltpu.VMEM((128, 16), jnp.float32),
            pltpu.VMEM_SHARED((_ACC_ROWS, 16), jnp.float32),
        ],
    )
    def k(sidx_hbm, z_hbm, ones_hbm, out_hbm, sidx_v, ones_v, cnt_sh):
        cid = lax.axis_index("c")
        tid = lax.axis_index("s")
        pltpu.sync_copy(z_hbm, cnt_sh.at[pl.ds(tid * _STRIPE, _STRIPE)])
        pltpu.sync_copy(ones_hbm, ones_v)
        plsc.subcore_barrier()

        def chunk(c, carry):
            r0 = cid * half + tid * (half // 16) + c * 8
            pltpu.sync_copy(sidx_hbm.at[pl.ds(r0, 8)], sidx_v)
            for j in range(8):
                pltpu.sync_copy(ones_v, cnt_sh.at[sidx_v.at[j]], add=True)
            return carry

        lax.fori_loop(0, half // 16 // 8, chunk, 0)
        plsc.subcore_barrier()
        pltpu.sync_copy(
            cnt_sh.at[pl.ds(tid * _STRIPE, _STRIPE)],
            out_hbm.at[pl.ds(cid * _ACC_ROWS + tid * _STRIPE, _STRIPE)],
        )

    return k(sidx, z_cnt, ones)


def _dense_layer(h, a0, a1, a2, cnt8, root, w, b2d, emit_stack=False):
    din = h.shape[1]
    br = 2000
    grid = (_N // br,)

    def body(h_ref, a0_ref, a1_ref, a2_ref, c_ref, root_ref, w_ref, b_ref,
             o_ref, *o2):
        acc = jnp.dot(h_ref[...], root_ref[...],
                      preferred_element_type=jnp.float32,
                      precision=lax.Precision.HIGHEST)
        for r, aref in enumerate((a0_ref, a1_ref, a2_ref)):
            recip = 1.0 / jnp.maximum(c_ref[:, r:r + 1], 1.0)
            acc = acc + jnp.dot(aref[...].astype(jnp.float32) * recip, w_ref[r],
                                preferred_element_type=jnp.float32,
                                precision=lax.Precision.HIGHEST)
        acc = jnp.maximum(acc + b_ref[...], 0.0)
        o_ref[...] = acc
        if emit_stack:
            for q in range(4):
                o2[0][q, :, :] = acc[:, 32 * q:32 * q + 32]

    node_spec = pl.BlockSpec((br, din), lambda i: (i, 0))
    out_specs = [pl.BlockSpec((br, 128), lambda i: (i, 0))]
    out_shape = [jax.ShapeDtypeStruct((_N, 128), jnp.float32)]
    if emit_stack:
        out_specs.append(pl.BlockSpec((4, br, 32), lambda i: (0, i, 0)))
        out_shape.append(jax.ShapeDtypeStruct((4, _N, 32), jnp.float32))
    res = pl.pallas_call(
        body,
        grid=grid,
        in_specs=[
            node_spec, node_spec, node_spec, node_spec,
            pl.BlockSpec((br, 8), lambda i: (i, 0)),
            pl.BlockSpec((din, 128), lambda i: (0, 0)),
            pl.BlockSpec((3, din, 128), lambda i: (0, 0, 0)),
            pl.BlockSpec((1, 128), lambda i: (0, 0)),
        ],
        out_specs=out_specs,
        out_shape=out_shape,
    )(h, a0, a1, a2, cnt8, root, w, b2d)
    return res if emit_stack else res[0]


def _pool_classify(h2, batch3d, wc, bc2d):
    br = 2000
    nsteps = _N // br

    def body(h_ref, b_ref, wc_ref, bc_ref, o_ref, sums, cnts):
        i = pl.program_id(0)

        @pl.when(i == 0)
        def _():
            sums[...] = jnp.zeros((_NUM_GRAPHS, 128), jnp.float32)
            cnts[...] = jnp.zeros((_NUM_GRAPHS, 8), jnp.float32)

        bb = b_ref[0, 0, :]
        onehot = (bb[:, None] == lax.broadcasted_iota(
            jnp.int32, (br, _NUM_GRAPHS), 1)).astype(jnp.float32)
        sums[...] += lax.dot_general(
            onehot, h_ref[...], (((0,), (0,)), ((), ())),
            preferred_element_type=jnp.float32,
            precision=lax.Precision.HIGHEST)
        cnts[:, 0:1] += lax.dot_general(
            onehot, jnp.ones((br, 1), jnp.float32), (((0,), (0,)), ((), ())),
            preferred_element_type=jnp.float32,
            precision=lax.Precision.HIGHEST)

        @pl.when(i == nsteps - 1)
        def _():
            recip = 1.0 / jnp.maximum(cnts[:, 0:1], 1.0)
            pooled = sums[...] * recip
            o_ref[...] = jnp.dot(pooled, wc_ref[...],
                                 preferred_element_type=jnp.float32,
                                 precision=lax.Precision.HIGHEST) + bc_ref[...]

    return pl.pallas_call(
        body,
        grid=(nsteps,),
        in_specs=[
            pl.BlockSpec((br, 128), lambda i: (i, 0)),
            pl.BlockSpec((1, 1, br), lambda i: (i, 0, 0)),
            pl.BlockSpec((128, 16), lambda i: (0, 0)),
            pl.BlockSpec((1, 16), lambda i: (0, 0)),
        ],
        out_specs=pl.BlockSpec((_NUM_GRAPHS, 16), lambda i: (0, 0)),
        out_shape=jax.ShapeDtypeStruct((_NUM_GRAPHS, 16), jnp.float32),
        scratch_shapes=[
            pltpu.VMEM((_NUM_GRAPHS, 128), jnp.float32),
            pltpu.VMEM((_NUM_GRAPHS, 8), jnp.float32),
        ],
    )(h2, batch3d, wc, bc2d)


def kernel(x, edge_index, edge_type, batch, shape_emb, color_emb, pos_emb,
           W1, root1, b1, W2, root2, b2, Wc, bc):
    tbl = jnp.concatenate([shape_emb, color_emb, pos_emb], axis=0)
    idx_flat = jnp.stack(
        [x[:, 0], x[:, 1] + 16, jnp.clip(x[:, 2], 0, 24) + 32], axis=1
    ).reshape(-1)
    idx_pad = jnp.concatenate(
        [idx_flat, jnp.zeros((_EMB_PAD - 3 * _N,), jnp.int32)]
    ).reshape(_EMB_PAD // 128, 128)
    e = _emb_gather(tbl, idx_pad)
    h0 = e[: 3 * _N].reshape(_N, 96)

    src = edge_index[0]
    dst = edge_index[1]
    pad = _EP - _E
    srcp = jnp.concatenate([src, jnp.zeros((pad,), jnp.int32)])
    sidx = jnp.concatenate(
        [edge_type * _N + dst, jnp.full((pad,), 3 * _N, jnp.int32)]
    ).reshape(_ROWS, 128)
    gidx = jnp.concatenate([srcp, srcp + _N]).reshape(2 * _ROWS, 128)

    z48 = jnp.zeros((_STRIPE, 48), jnp.float32)
    z16 = jnp.zeros((_STRIPE, 16), jnp.float32)
    ones = jnp.ones((128, 16), jnp.float32)

    cnt_raw = _edge_counts(sidx, z16, ones)
    cnt_sum = cnt_raw[:_ACC_ROWS] + cnt_raw[_ACC_ROWS:]
    cnt = cnt_sum[: 3 * _N, 0].reshape(3, _N)
    cnt8 = jnp.pad(cnt.T, ((0, 0), (0, 5)))

    h0_stack = h0.reshape(_N, 2, 48).transpose(1, 0, 2).reshape(2 * _N, 48)
    a1_raw = _edge_agg(h0_stack, gidx, sidx, z48, 48, 5)
    a1 = (a1_raw.reshape(2, _ACC_ROWS, 48)[:, : 3 * _N]
          .reshape(2, 3, _N, 48).transpose(1, 2, 0, 3).reshape(3, _N, 96))
    h1, h1_stack = _dense_layer(h0, a1[0], a1[1], a1[2], cnt8, root1, W1,
                                b1.reshape(1, 128), emit_stack=True)

    z32 = jnp.zeros((_STRIPE, 32), jnp.float32)
    stack4 = h1_stack.reshape(4 * _N, 32)
    a2a_raw = _edge_agg(stack4[: 2 * _N], gidx, sidx, z32, 32, 16)
    a2b_raw = _edge_agg(stack4[2 * _N:], gidx, sidx, z32, 32, 16)
    a2a = (a2a_raw.reshape(2, _ACC_ROWS, 32)[:, : 3 * _N]
           .reshape(2, 3, _N, 32).transpose(1, 2, 0, 3).reshape(3, _N, 64))
    a2b = (a2b_raw.reshape(2, _ACC_ROWS, 32)[:, : 3 * _N]
           .reshape(2, 3, _N, 32).transpose(1, 2, 0, 3).reshape(3, _N, 64))
    a2 = jnp.concatenate([a2a, a2b], axis=-1)
    h2 = _dense_layer(h1, a2[0], a2[1], a2[2], cnt8, root2, W2,
                      b2.reshape(1, 128))

    return _pool_classify(h2, batch.reshape(_N // 2000, 1, 2000), Wc,
                          bc.reshape(1, 16))


# speedup vs baseline: 2.0534x; 1.0002x over previous
import functools

import jax
import jax.numpy as jnp
from jax import lax
from jax.experimental import pallas as pl
from jax.experimental.pallas import tpu as pltpu
from jax.experimental.pallas import tpu_sc as plsc

_N = 10000
_E = 320000
_NUM_GRAPHS = 256
_NUM_REL = 3
_EP = 327680
_ROWS = _EP // 128
_ACC_ROWS = 30080
_STRIPE = _ACC_ROWS // 16
_EMB_PAD = 32768


def _emb_gather(tbl, idx2d):
    mesh = plsc.VectorSubcoreMesh(core_axis_name="c", subcore_axis_name="s")

    @functools.partial(
        pl.kernel,
        out_type=jax.ShapeDtypeStruct((_EMB_PAD, 32), jnp.float32),
        mesh=mesh,
        compiler_params=pltpu.CompilerParams(use_tc_tiling_on_sc=False),
        scratch_types=[
            pltpu.VMEM((8, 128), jnp.int32),
            pltpu.VMEM((1024, 32), jnp.float32),
            pltpu.SemaphoreType.DMA,
        ],
    )
    def k(tbl_hbm, idx_hbm, out_hbm, idx_v, rows_v, sem):
        cid = lax.axis_index("c")
        sid = lax.axis_index("s")
        wid = sid * 2 + cid
        pltpu.sync_copy(idx_hbm.at[pl.ds(wid * 8, 8)], idx_v)
        gathers = [
            pltpu.async_copy(
                tbl_hbm.at[idx_v.at[j]], rows_v.at[pl.ds(j * 128, 128)], sem
            )
            for j in range(8)
        ]
        for d in gathers:
            d.wait()
        pltpu.sync_copy(rows_v, out_hbm.at[pl.ds(wid * 1024, 1024)])

    return k(tbl, idx2d)


def _edge_agg(h_stack, gidx, sidx, z_acc, dh, krows):
    mesh = plsc.VectorSubcoreMesh(core_axis_name="c", subcore_axis_name="s")

    @functools.partial(
        pl.kernel,
        out_type=jax.ShapeDtypeStruct((2 * _ACC_ROWS, dh), jnp.float32),
        mesh=mesh,
        compiler_params=pltpu.CompilerParams(use_tc_tiling_on_sc=False),
        scratch_types=[
            pltpu.VMEM((krows, 128), jnp.int32),
            pltpu.VMEM((krows, 128), jnp.int32),
            pltpu.VMEM((krows * 128, dh), jnp.float32),
            pltpu.VMEM_SHARED((_ACC_ROWS, dh), jnp.float32),
            pltpu.SemaphoreType.DMA,
            pltpu.SemaphoreType.DMA,
        ],
    )
    def k(h_hbm, gidx_hbm, sidx_hbm, z_hbm,
          out_hbm, sidx_v, gidx_v, rows_v, acc_sh, sem, sem2):
        cid = lax.axis_index("c")
        tid = lax.axis_index("s")

        pltpu.sync_copy(z_hbm, acc_sh.at[pl.ds(tid * _STRIPE, _STRIPE)])
        plsc.subcore_barrier()

        def chunk(c, carry):
            r0 = tid * 160 + c * krows
            pltpu.sync_copy(sidx_hbm.at[pl.ds(r0, krows)], sidx_v)
            pltpu.sync_copy(gidx_hbm.at[pl.ds(cid * _ROWS + r0, krows)], gidx_v)
            gathers = [
                pltpu.async_copy(
                    h_hbm.at[gidx_v.at[j]],
                    rows_v.at[pl.ds(j * 128, 128)],
                    sem,
                )
                for j in range(krows)
            ]
            scatters = []
            for j in range(krows):
                gathers[j].wait()
                scatters.append(pltpu.async_copy(
                    rows_v.at[pl.ds(j * 128, 128)],
                    acc_sh.at[sidx_v.at[j]],
                    sem2,
                    add=True,
                ))
            for d in scatters:
                d.wait()
            return carry

        lax.fori_loop(0, 160 // krows, chunk, 0)
        plsc.subcore_barrier()

        pltpu.sync_copy(
            acc_sh.at[pl.ds(tid * _STRIPE, _STRIPE)],
            out_hbm.at[pl.ds(cid * _ACC_ROWS + tid * _STRIPE, _STRIPE)],
        )

    return k(h_stack, gidx, sidx, z_acc)


def _edge_counts(sidx, z_cnt, ones):
    mesh = plsc.VectorSubcoreMesh(core_axis_name="c", subcore_axis_name="s")
    half = _ROWS // 2

    @functools.partial(
        pl.kernel,
        out_type=jax.ShapeDtypeStruct((2 * _ACC_ROWS, 16), jnp.float32),
        mesh=mesh,
        compiler_params=pltpu.CompilerParams(use_tc_tiling_on_sc=False),
        scratch_types=[
            pltpu.VMEM((8, 128), jnp.int32),
            pltpu.VMEM((128, 16), jnp.float32),
            pltpu.VMEM_SHARED((_ACC_ROWS, 16), jnp.float32),
        ],
    )
    def k(sidx_hbm, z_hbm, ones_hbm, out_hbm, sidx_v, ones_v, cnt_sh):
        cid = lax.axis_index("c")
        tid = lax.axis_index("s")
        pltpu.sync_copy(z_hbm, cnt_sh.at[pl.ds(tid * _STRIPE, _STRIPE)])
        pltpu.sync_copy(ones_hbm, ones_v)
        plsc.subcore_barrier()

        def chunk(c, carry):
            r0 = cid * half + tid * (half // 16) + c * 8
            pltpu.sync_copy(sidx_hbm.at[pl.ds(r0, 8)], sidx_v)
            for j in range(8):
                pltpu.sync_copy(ones_v, cnt_sh.at[sidx_v.at[j]], add=True)
            return carry

        lax.fori_loop(0, half // 16 // 8, chunk, 0)
        plsc.subcore_barrier()
        pltpu.sync_copy(
            cnt_sh.at[pl.ds(tid * _STRIPE, _STRIPE)],
            out_hbm.at[pl.ds(cid * _ACC_ROWS + tid * _STRIPE, _STRIPE)],
        )

    return k(sidx, z_cnt, ones)


def _dense_layer(h, a0, a1, a2, cnt8, root, w, b2d, emit_stack=False):
    din = h.shape[1]
    br = 2000
    grid = (_N // br,)

    def body(h_ref, a0_ref, a1_ref, a2_ref, c_ref, root_ref, w_ref, b_ref,
             o_ref, *o2):
        acc = jnp.dot(h_ref[...], root_ref[...],
                      preferred_element_type=jnp.float32,
                      precision=lax.Precision.HIGHEST)
        for r, aref in enumerate((a0_ref, a1_ref, a2_ref)):
            recip = 1.0 / jnp.maximum(c_ref[:, r:r + 1], 1.0)
            acc = acc + jnp.dot(aref[...].astype(jnp.float32) * recip, w_ref[r],
                                preferred_element_type=jnp.float32,
                                precision=lax.Precision.HIGHEST)
        acc = jnp.maximum(acc + b_ref[...], 0.0)
        o_ref[...] = acc
        if emit_stack:
            for q in range(4):
                o2[0][q, :, :] = acc[:, 32 * q:32 * q + 32]

    node_spec = pl.BlockSpec((br, din), lambda i: (i, 0))
    out_specs = [pl.BlockSpec((br, 128), lambda i: (i, 0))]
    out_shape = [jax.ShapeDtypeStruct((_N, 128), jnp.float32)]
    if emit_stack:
        out_specs.append(pl.BlockSpec((4, br, 32), lambda i: (0, i, 0)))
        out_shape.append(jax.ShapeDtypeStruct((4, _N, 32), jnp.float32))
    res = pl.pallas_call(
        body,
        grid=grid,
        in_specs=[
            node_spec, node_spec, node_spec, node_spec,
            pl.BlockSpec((br, 8), lambda i: (i, 0)),
            pl.BlockSpec((din, 128), lambda i: (0, 0)),
            pl.BlockSpec((3, din, 128), lambda i: (0, 0, 0)),
            pl.BlockSpec((1, 128), lambda i: (0, 0)),
        ],
        out_specs=out_specs,
        out_shape=out_shape,
    )(h, a0, a1, a2, cnt8, root, w, b2d)
    return res if emit_stack else res[0]


def _pool_classify(h2, batch3d, wc, bc2d):
    br = 2000
    nsteps = _N // br

    def body(h_ref, b_ref, wc_ref, bc_ref, o_ref, sums, cnts):
        i = pl.program_id(0)

        @pl.when(i == 0)
        def _():
            sums[...] = jnp.zeros((_NUM_GRAPHS, 128), jnp.float32)
            cnts[...] = jnp.zeros((_NUM_GRAPHS, 8), jnp.float32)

        bb = b_ref[0, 0, :]
        onehot = (bb[:, None] == lax.broadcasted_iota(
            jnp.int32, (br, _NUM_GRAPHS), 1)).astype(jnp.float32)
        sums[...] += lax.dot_general(
            onehot, h_ref[...], (((0,), (0,)), ((), ())),
            preferred_element_type=jnp.float32,
            precision=lax.Precision.HIGHEST)
        cnts[:, 0:1] += lax.dot_general(
            onehot, jnp.ones((br, 1), jnp.float32), (((0,), (0,)), ((), ())),
            preferred_element_type=jnp.float32,
            precision=lax.Precision.HIGHEST)

        @pl.when(i == nsteps - 1)
        def _():
            recip = 1.0 / jnp.maximum(cnts[:, 0:1], 1.0)
            pooled = sums[...] * recip
            o_ref[...] = jnp.dot(pooled, wc_ref[...],
                                 preferred_element_type=jnp.float32,
                                 precision=lax.Precision.HIGHEST) + bc_ref[...]

    return pl.pallas_call(
        body,
        grid=(nsteps,),
        in_specs=[
            pl.BlockSpec((br, 128), lambda i: (i, 0)),
            pl.BlockSpec((1, 1, br), lambda i: (i, 0, 0)),
            pl.BlockSpec((128, 16), lambda i: (0, 0)),
            pl.BlockSpec((1, 16), lambda i: (0, 0)),
        ],
        out_specs=pl.BlockSpec((_NUM_GRAPHS, 16), lambda i: (0, 0)),
        out_shape=jax.ShapeDtypeStruct((_NUM_GRAPHS, 16), jnp.float32),
        scratch_shapes=[
            pltpu.VMEM((_NUM_GRAPHS, 128), jnp.float32),
            pltpu.VMEM((_NUM_GRAPHS, 8), jnp.float32),
        ],
    )(h2, batch3d, wc, bc2d)


def kernel(x, edge_index, edge_type, batch, shape_emb, color_emb, pos_emb,
           W1, root1, b1, W2, root2, b2, Wc, bc):
    tbl = jnp.concatenate([shape_emb, color_emb, pos_emb], axis=0)
    idx_flat = jnp.stack(
        [x[:, 0], x[:, 1] + 16, jnp.clip(x[:, 2], 0, 24) + 32], axis=1
    ).reshape(-1)
    idx_pad = jnp.concatenate(
        [idx_flat, jnp.zeros((_EMB_PAD - 3 * _N,), jnp.int32)]
    ).reshape(_EMB_PAD // 128, 128)
    e = _emb_gather(tbl, idx_pad)
    h0 = e[: 3 * _N].reshape(_N, 96)

    src = edge_index[0]
    dst = edge_index[1]
    pad = _EP - _E
    srcp = jnp.concatenate([src, jnp.zeros((pad,), jnp.int32)])
    sidx = jnp.concatenate(
        [edge_type * _N + dst, jnp.full((pad,), 3 * _N, jnp.int32)]
    ).reshape(_ROWS, 128)
    gidx = jnp.concatenate([srcp, srcp + _N]).reshape(2 * _ROWS, 128)

    z48 = jnp.zeros((_STRIPE, 48), jnp.float32)
    z16 = jnp.zeros((_STRIPE, 16), jnp.float32)
    ones = jnp.ones((128, 16), jnp.float32)

    cnt_raw = _edge_counts(sidx, z16, ones)
    cnt_sum = cnt_raw[:_ACC_ROWS] + cnt_raw[_ACC_ROWS:]
    cnt = cnt_sum[: 3 * _N, 0].reshape(3, _N)
    cnt8 = jnp.pad(cnt.T, ((0, 0), (0, 5)))

    h0_stack = h0.reshape(_N, 2, 48).transpose(1, 0, 2).reshape(2 * _N, 48)
    a1_raw = _edge_agg(h0_stack, gidx, sidx, z48, 48, 5)
    a1 = (a1_raw.reshape(2, _ACC_ROWS, 48)[:, : 3 * _N]
          .reshape(2, 3, _N, 48).transpose(1, 2, 0, 3).reshape(3, _N, 96))
    h1, h1_stack = _dense_layer(h0, a1[0], a1[1], a1[2], cnt8, root1, W1,
                                b1.reshape(1, 128), emit_stack=True)

    z32 = jnp.zeros((_STRIPE, 32), jnp.float32)
    stack4 = h1_stack.reshape(4 * _N, 32)
    a2a_raw = _edge_agg(stack4[: 2 * _N], gidx, sidx, z32, 32, 16)
    a2b_raw = _edge_agg(stack4[2 * _N:], gidx, sidx, z32, 32, 16)
    a2a = (a2a_raw.reshape(2, _ACC_ROWS, 32)[:, : 3 * _N]
           .reshape(2, 3, _N, 32).transpose(1, 2, 0, 3).reshape(3, _N, 64))
    a2b = (a2b_raw.reshape(2, _ACC_ROWS, 32)[:, : 3 * _N]
           .reshape(2, 3, _N, 32).transpose(1, 2, 0, 3).reshape(3, _N, 64))
    a2 = jnp.concatenate([a2a, a2b], axis=-1)
    h2 = _dense_layer(h1, a2[0], a2[1], a2[2], cnt8, root2, W2,
                      b2.reshape(1, 128))

    return _pool_classify(h2, batch.reshape(_N // 2000, 1, 2000), Wc,
                          bc.reshape(1, 16))
